# baseline (speedup 1.0000x reference)
import jax
import jax.numpy as jnp
from jax.experimental import pallas as pl


def _copy_body(xu_ref, xi_ref, ou_ref, oi_ref):
    ou_ref[...] = xu_ref[...]
    oi_ref[...] = xi_ref[...]


def kernel(x_user, x_item, edge_index):
    del edge_index
    n, d = x_user.shape
    blk = 2000
    grid = (n // blk,)
    spec = pl.BlockSpec((blk, d), lambda i: (i, 0))
    ou, oi = pl.pallas_call(
        _copy_body,
        grid=grid,
        in_specs=[spec, spec],
        out_specs=[spec, spec],
        out_shape=[
            jax.ShapeDtypeStruct((n, d), x_user.dtype),
            jax.ShapeDtypeStruct((n, d), x_item.dtype),
        ],
    )(x_user, x_item)
    return (ou, oi)
